# baseline (speedup 1.0000x reference)
import jax
import jax.numpy as jnp
from jax.experimental import pallas as pl

N = 10000
H = 4
C = 32


def _bias_add(x_ref, b_ref, o_ref):
    o_ref[...] = x_ref[...] + b_ref[...]


def kernel(x, edge_index, edge_weight, W_l, b_l, W_r, b_r, W_e, att, bias):
    src, dst = edge_index[0], edge_index[1]
    loop_w = jax.ops.segment_max(edge_weight, dst, num_segments=N)
    loop_w = jnp.where(jnp.isfinite(loop_w), loop_w, 0.0)
    loop_idx = jnp.arange(N, dtype=src.dtype)
    src_full = jnp.concatenate([src, loop_idx])
    dst_full = jnp.concatenate([dst, loop_idx])
    ea_full = jnp.concatenate([edge_weight[:, None], loop_w[:, None]], axis=0)

    x_l = (x @ W_l + b_l).reshape(N, H, C)
    x_r = (x @ W_r + b_r).reshape(N, H, C)
    e = (ea_full @ W_e).reshape(-1, H, C)

    xj = x_l[src_full]
    xi = x_r[dst_full]
    z = jax.nn.leaky_relu(xj + xi + e, 0.2)
    alpha = (z * att[None]).sum(-1)
    amax = jax.ops.segment_max(alpha, dst_full, num_segments=N)
    amax = jnp.where(jnp.isfinite(amax), amax, 0.0)
    alpha = jnp.exp(alpha - amax[dst_full])
    denom = jax.ops.segment_sum(alpha, dst_full, num_segments=N)
    alpha = alpha / (denom[dst_full] + 1e-16)
    msg = xj * alpha[..., None]
    out = jax.ops.segment_sum(msg, dst_full, num_segments=N)
    out = out.reshape(N, H * C)

    return pl.pallas_call(
        _bias_add,
        out_shape=jax.ShapeDtypeStruct((N, H * C), jnp.float32),
        grid=(10,),
        in_specs=[
            pl.BlockSpec((N // 10, H * C), lambda i: (i, 0)),
            pl.BlockSpec((1, H * C), lambda i: (0, 0)),
        ],
        out_specs=pl.BlockSpec((N // 10, H * C), lambda i: (i, 0)),
    )(out, bias.reshape(1, H * C))


# speedup vs baseline: 30.7490x; 30.7490x over previous
import functools

import jax
import jax.numpy as jnp
from jax import lax
from jax.experimental import pallas as pl
from jax.experimental.pallas import tpu as pltpu
from jax.experimental.pallas import tpu_sc as plsc

N = 10000
E = 320000
D = 128
H = 4
C = 32
HC = H * C

NC = 2
NS = 16
NW = NC * NS
EPW = E // NW
B = 80
NB = EPW // B
NP = 10240
NA = 8192
NBR = 2048
ND8 = NP // 8

_SLOPE = 0.2


def _proj_body(x_ref, wl_ref, wr_ref, bl_ref, br_ref, xl_ref, xr_ref):
    xv = x_ref[...]
    xl_ref[...] = jnp.dot(xv, wl_ref[...], preferred_element_type=jnp.float32) + bl_ref[...]
    xr_ref[...] = jnp.dot(xv, wr_ref[...], preferred_element_type=jnp.float32) + br_ref[...]


def _project(x, W_l, b_l, W_r, b_r):
    blk = 1000
    return pl.pallas_call(
        _proj_body,
        grid=(N // blk,),
        in_specs=[
            pl.BlockSpec((blk, D), lambda i: (i, 0)),
            pl.BlockSpec((D, HC), lambda i: (0, 0)),
            pl.BlockSpec((D, HC), lambda i: (0, 0)),
            pl.BlockSpec((1, HC), lambda i: (0, 0)),
            pl.BlockSpec((1, HC), lambda i: (0, 0)),
        ],
        out_specs=[
            pl.BlockSpec((blk, HC), lambda i: (i, 0)),
            pl.BlockSpec((blk, HC), lambda i: (i, 0)),
        ],
        out_shape=[
            jax.ShapeDtypeStruct((N, HC), jnp.float32),
            jax.ShapeDtypeStruct((N, HC), jnp.float32),
        ],
    )(x, W_l, W_r, b_l.reshape(1, HC), b_r.reshape(1, HC))


def _sca_body(src_h, dst_h, w_h, xl_h, xr_h, we_h, att_h,
              acc_o, lw_o, ea_o,
              acc_sp,
              src_sc, dst_sc, w_sc, ia_sc, xl_rows, xr_rows, m_v, ea_v,
              lw_t, zb, wa_v, sem, sem2, sem3):
    cid = lax.axis_index("c")
    sid = lax.axis_index("s")
    wid = cid * NS + sid

    zero16 = jnp.zeros((16,), jnp.float32)

    def z1(t, _):
        zb[t // 8, pl.ds((t % 8) * 16, 16)] = zero16
        return 0
    lax.fori_loop(0, 32 * 8, z1, 0)

    def z2(k, _):
        lw_t[pl.ds(k * 16, 16)] = zero16
        return 0
    lax.fori_loop(0, NP // 16, z2, 0)

    row0 = pl.multiple_of(sid * (NA // NS), 8)
    for k in range(16):
        pltpu.sync_copy(zb, acc_sp.at[pl.ds(row0 + k * 32, 32)])

    pltpu.sync_copy(we_h, wa_v.at[0])
    pltpu.sync_copy(att_h, wa_v.at[1])

    plsc.subcore_barrier()

    wev = tuple(wa_v[0, pl.ds(v * 16, 16)] for v in range(8))
    atv = tuple(wa_v[1, pl.ds(v * 16, 16)] for v in range(8))
    lane = lax.iota(jnp.int32, 16)
    lane0 = lane == 0

    def batch_body(j, carry):
        wev, atv, lane0 = carry
        base = pl.multiple_of(wid * EPW + j * B, 8)
        i1 = pltpu.async_copy(src_h.at[pl.ds(base, B)], src_sc, sem3)
        i2 = pltpu.async_copy(dst_h.at[pl.ds(base, B)], dst_sc, sem3)
        i3 = pltpu.async_copy(w_h.at[pl.ds(base, B)], w_sc, sem3)
        i1.wait()
        i2.wait()
        i3.wait()
        g1 = pltpu.async_copy(xl_h.at[src_sc], xl_rows, sem)
        g2 = pltpu.async_copy(xr_h.at[dst_sc], xr_rows, sem2)
        g1.wait()
        g2.wait()

        def group_body(g, carry):
            wev, atv, lane0 = carry
            d16 = dst_sc[pl.ds(g * 16, 16)]
            w16 = w_sc[pl.ds(g * 16, 16)]
            ia_sc[pl.ds(g * 16, 16)] = jnp.where(d16 < NA, d16, 0)
            for k in range(16):
                dkv = jnp.full((16,), d16[k], jnp.int32)
                wkv = jnp.full((16,), w16[k], jnp.float32)
                cur = plsc.load_gather(lw_t, [dkv])
                plsc.store_scatter(lw_t, [dkv], jnp.maximum(cur, wkv),
                                   mask=lane0)
            for k in range(16):
                i = g * 16 + k
                w_s = w16[k]
                ind = (d16[k] < NA).astype(jnp.float32)
                ps = []
                for v in range(8):
                    xlv = xl_rows[i, pl.ds(v * 16, 16)]
                    xrv = xr_rows[i, pl.ds(v * 16, 16)]
                    u = xlv + xrv + w_s * wev[v]
                    t = jnp.maximum(u, _SLOPE * u) * atv[v]
                    ps.append((t, xlv))
                evecs = []
                for h in range(H):
                    a_h = jnp.sum(ps[2 * h][0] + ps[2 * h + 1][0])
                    e_h = jnp.exp(jnp.full((16,), a_h))
                    evecs.append(e_h)
                    ea_v[pl.ds(i * 64 + h * 16, 16)] = e_h
                for v in range(8):
                    m_v[i, pl.ds(v * 16, 16)] = ps[v][1] * (evecs[v // 2] * ind)
            return (wev, atv, lane0)

        carry = lax.fori_loop(0, B // 16, group_body, carry)

        s1 = pltpu.async_copy(m_v, acc_sp.at[ia_sc], sem, add=True)
        ear = pl.multiple_of((wid * NB + j) * B * 64, 8)
        s2 = pltpu.async_copy(ea_v, ea_o.at[pl.ds(ear, B * 64)], sem2)
        s1.wait()
        s2.wait()
        return carry

    lax.fori_loop(0, NB, batch_body, (wev, atv, lane0))

    plsc.subcore_barrier()

    orow = pl.multiple_of(cid * NA + sid * (NA // NS), 8)
    pltpu.sync_copy(acc_sp.at[pl.ds(row0, NA // NS)],
                    acc_o.at[pl.ds(orow, NA // NS)])
    pltpu.sync_copy(lw_t, lw_o.at[pl.ds(pl.multiple_of(wid * NP, 8), NP)])


def _sca(src1, dst1, w1, xl, xr, we_row, att_row):
    mesh = plsc.VectorSubcoreMesh(core_axis_name="c", subcore_axis_name="s")
    f = functools.partial(
        pl.kernel,
        out_type=[
            jax.ShapeDtypeStruct((NC * NA, HC), jnp.float32),
            jax.ShapeDtypeStruct((NW * NP,), jnp.float32),
            jax.ShapeDtypeStruct((E * 64,), jnp.float32),
        ],
        mesh=mesh,
        compiler_params=pltpu.CompilerParams(needs_layout_passes=False),
        scratch_types=[
            pltpu.VMEM_SHARED((NA, HC), jnp.float32),
            pltpu.VMEM((B,), jnp.int32),
            pltpu.VMEM((B,), jnp.int32),
            pltpu.VMEM((B,), jnp.float32),
            pltpu.VMEM((B,), jnp.int32),
            pltpu.VMEM((B, HC), jnp.float32),
            pltpu.VMEM((B, HC), jnp.float32),
            pltpu.VMEM((B, HC), jnp.float32),
            pltpu.VMEM((B * 64,), jnp.float32),
            pltpu.VMEM((NP,), jnp.float32),
            pltpu.VMEM((32, HC), jnp.float32),
            pltpu.VMEM((2, HC), jnp.float32),
            pltpu.SemaphoreType.DMA,
            pltpu.SemaphoreType.DMA,
            pltpu.SemaphoreType.DMA,
        ],
    )(_sca_body)
    return f(src1, dst1, w1, xl, xr, we_row, att_row)


def _scb_body(src_h, dst_h, xl_h, ea_h,
              acc_o, den_o,
              acc_sp, den_sp,
              src_sc, dst_sc, ib_sc, dn_sc, xl_rows, m_v, dp_v, ea_v, zb,
              sem, sem2, sem3):
    cid = lax.axis_index("c")
    sid = lax.axis_index("s")
    wid = cid * NS + sid

    zero16 = jnp.zeros((16,), jnp.float32)

    def z1(t, _):
        zb[t // 8, pl.ds((t % 8) * 16, 16)] = zero16
        return 0
    lax.fori_loop(0, 32 * 8, z1, 0)

    rowb = pl.multiple_of(sid * (NBR // NS), 8)
    for k in range(4):
        pltpu.sync_copy(zb, acc_sp.at[pl.ds(rowb + k * 32, 32)])
    rowd = pl.multiple_of(sid * (ND8 // NS), 8)
    for k in range(2):
        pltpu.sync_copy(zb, den_sp.at[pl.ds(rowd + k * 32, 32)])
    pltpu.sync_copy(zb.at[pl.ds(0, 16)], den_sp.at[pl.ds(rowd + 64, 16)])

    plsc.subcore_barrier()

    lane = lax.iota(jnp.int32, 16)

    def batch_body(j, carry):
        lane, = carry
        base = pl.multiple_of(wid * EPW + j * B, 8)
        i1 = pltpu.async_copy(src_h.at[pl.ds(base, B)], src_sc, sem3)
        i2 = pltpu.async_copy(dst_h.at[pl.ds(base, B)], dst_sc, sem3)
        i1.wait()
        i2.wait()
        g1 = pltpu.async_copy(xl_h.at[src_sc], xl_rows, sem)
        ear = pl.multiple_of((wid * NB + j) * B * 64, 8)
        g2 = pltpu.async_copy(ea_h.at[pl.ds(ear, B * 64)], ea_v, sem2)
        g1.wait()
        g2.wait()

        def group_body(g, carry):
            lane, = carry
            d16 = dst_sc[pl.ds(g * 16, 16)]
            ib_sc[pl.ds(g * 16, 16)] = jnp.where(d16 >= NA, d16 - NA, 0)
            dn_sc[pl.ds(g * 16, 16)] = d16 // 8
            for k in range(16):
                i = g * 16 + k
                ind = (d16[k] >= NA).astype(jnp.float32)
                m8 = (d16[k] % 8) * 16
                pack = jnp.zeros((16,), jnp.float32)
                for h in range(H):
                    e_h = ea_v[pl.ds(i * 64 + h * 16, 16)]
                    pack = pack + e_h * (lane == h).astype(jnp.float32)
                    sl = e_h * ind
                    m_v[i, pl.ds(h * 32, 16)] = xl_rows[i, pl.ds(h * 32, 16)] * sl
                    m_v[i, pl.ds(h * 32 + 16, 16)] = (
                        xl_rows[i, pl.ds(h * 32 + 16, 16)] * sl)
                zero16v = jnp.zeros((16,), jnp.float32)
                for q in range(8):
                    dp_v[i, pl.ds(q * 16, 16)] = zero16v
                dp_v[i, pl.ds(m8, 16)] = pack
            return (lane,)

        carry = lax.fori_loop(0, B // 16, group_body, carry)

        s1 = pltpu.async_copy(m_v, acc_sp.at[ib_sc], sem, add=True)
        s2 = pltpu.async_copy(dp_v, den_sp.at[dn_sc], sem2, add=True)
        s1.wait()
        s2.wait()
        return carry

    lax.fori_loop(0, NB, batch_body, (lane,))

    plsc.subcore_barrier()

    orow = pl.multiple_of(cid * NBR + sid * (NBR // NS), 8)
    pltpu.sync_copy(acc_sp.at[pl.ds(rowb, NBR // NS)],
                    acc_o.at[pl.ds(orow, NBR // NS)])
    drow = pl.multiple_of(cid * ND8 + sid * (ND8 // NS), 8)
    pltpu.sync_copy(den_sp.at[pl.ds(rowd, ND8 // NS)],
                    den_o.at[pl.ds(drow, ND8 // NS)])


def _scb(src1, dst1, xl, ea):
    mesh = plsc.VectorSubcoreMesh(core_axis_name="c", subcore_axis_name="s")
    f = functools.partial(
        pl.kernel,
        out_type=[
            jax.ShapeDtypeStruct((NC * NBR, HC), jnp.float32),
            jax.ShapeDtypeStruct((NC * ND8, HC), jnp.float32),
        ],
        mesh=mesh,
        compiler_params=pltpu.CompilerParams(needs_layout_passes=False),
        scratch_types=[
            pltpu.VMEM_SHARED((NBR, HC), jnp.float32),
            pltpu.VMEM_SHARED((ND8, HC), jnp.float32),
            pltpu.VMEM((B,), jnp.int32),
            pltpu.VMEM((B,), jnp.int32),
            pltpu.VMEM((B,), jnp.int32),
            pltpu.VMEM((B,), jnp.int32),
            pltpu.VMEM((B, HC), jnp.float32),
            pltpu.VMEM((B, HC), jnp.float32),
            pltpu.VMEM((B, HC), jnp.float32),
            pltpu.VMEM((B * 64,), jnp.float32),
            pltpu.VMEM((32, HC), jnp.float32),
            pltpu.SemaphoreType.DMA,
            pltpu.SemaphoreType.DMA,
            pltpu.SemaphoreType.DMA,
        ],
    )(_scb_body)
    return f(src1, dst1, xl, ea)


def _final_body(a0_ref, a1_ref, d0_ref, d1_ref, lw_ref,
                xl_ref, xr_ref, we_ref, attb_ref, b4_ref, bias_ref, o_ref):
    xl = xl_ref[...]
    lw = jnp.max(lw_ref[...], axis=1)[:, None]
    u = xl + xr_ref[...] + lw * we_ref[...]
    t = jnp.maximum(u, _SLOPE * u)
    eb = jnp.exp(jnp.dot(t, attb_ref[...], preferred_element_type=jnp.float32))
    den_b = jnp.dot(d0_ref[...] + d1_ref[...], b4_ref[...],
                    preferred_element_type=jnp.float32) + eb
    num = a0_ref[...] + a1_ref[...] + xl * eb
    o_ref[...] = num / (den_b + 1e-16) + bias_ref[...]


def _finalize(nrows, acc2, den8, lw_nt, xl, xr, we_row, att_bcast, b4, bias):
    blk = 512
    g = nrows // blk
    return pl.pallas_call(
        _final_body,
        grid=(g,),
        in_specs=[
            pl.BlockSpec((blk, HC), lambda i: (i, 0)),
            pl.BlockSpec((blk, HC), lambda i, g=g: (i + g, 0)),
            pl.BlockSpec((blk, 8), lambda i: (i, 0)),
            pl.BlockSpec((blk, 8), lambda i, g=g: (i + g, 0)),
            pl.BlockSpec((blk, NW), lambda i: (i, 0)),
            pl.BlockSpec((blk, HC), lambda i: (i, 0)),
            pl.BlockSpec((blk, HC), lambda i: (i, 0)),
            pl.BlockSpec((1, HC), lambda i: (0, 0)),
            pl.BlockSpec((HC, HC), lambda i: (0, 0)),
            pl.BlockSpec((8, HC), lambda i: (0, 0)),
            pl.BlockSpec((1, HC), lambda i: (0, 0)),
        ],
        out_specs=pl.BlockSpec((blk, HC), lambda i: (i, 0)),
        out_shape=jax.ShapeDtypeStruct((nrows, HC), jnp.float32),
    )(acc2, acc2, den8, den8, lw_nt, xl, xr,
      we_row.reshape(1, HC), att_bcast, b4, bias.reshape(1, HC))


def kernel(x, edge_index, edge_weight, W_l, b_l, W_r, b_r, W_e, att, bias):
    src = edge_index[0]
    dst = edge_index[1]
    w1 = edge_weight

    xl, xr = _project(x, W_l, b_l, W_r, b_r)

    we_row = W_e.reshape(HC)
    att_flat = att.reshape(HC)
    accA, lw_flat, ea = _sca(src, dst, w1, xl, xr, we_row, att_flat)
    accB, den8 = _scb(src, dst, xl, ea)

    ci = jnp.arange(HC, dtype=jnp.int32) // C
    att_bcast = jnp.where(ci[:, None] == ci[None, :], att_flat[:, None], 0.0)
    hh = jnp.arange(8, dtype=jnp.int32)
    b4 = (hh[:, None] == ci[None, :]).astype(jnp.float32)

    den4 = den8.reshape(NC * ND8, 8, 16)[:, :, :4]
    den4 = den4.reshape(NC, NP, 4)
    den4 = jnp.pad(den4, ((0, 0), (0, 0), (0, 4)))

    lw_nt = lw_flat.reshape(NW, NP)[:, :N].T
    lw_pad = jnp.pad(lw_nt, ((0, NP - N), (0, 0)))
    xl_pad = jnp.pad(xl, ((0, NP - N), (0, 0)))
    xr_pad = jnp.pad(xr, ((0, NP - N), (0, 0)))

    denA = jnp.concatenate([den4[0, :NA], den4[1, :NA]], 0)
    outA = _finalize(NA, accA, denA, lw_pad[:NA], xl_pad[:NA], xr_pad[:NA],
                     we_row, att_bcast, b4, bias)

    denB = jnp.concatenate([den4[0, NA:], den4[1, NA:]], 0)
    outB = _finalize(NBR, accB, denB, lw_pad[NA:], xl_pad[NA:], xr_pad[NA:],
                     we_row, att_bcast, b4, bias)

    return jnp.concatenate([outA, outB[:N - NA]], axis=0)


# speedup vs baseline: 39.9772x; 1.3001x over previous
import functools

import jax
import jax.numpy as jnp
from jax import lax
from jax.experimental import pallas as pl
from jax.experimental.pallas import tpu as pltpu
from jax.experimental.pallas import tpu_sc as plsc

N = 10000
E = 320000
D = 128
H = 4
C = 32
HC = H * C

NC = 2
NS = 16
NW = NC * NS
EPW = E // NW
B = 80
NB = EPW // B
BA = 80
NBA = EPW // BA
NP = 10240
NA = 4096
NBR = 6144
ND8 = NP // 8

_SLOPE = 0.2


def _proj_body(x_ref, wl_ref, wr_ref, bl_ref, br_ref, xl_ref, xr_ref):
    xv = x_ref[...]
    xl_ref[...] = jnp.dot(xv, wl_ref[...], preferred_element_type=jnp.float32) + bl_ref[...]
    xr_ref[...] = jnp.dot(xv, wr_ref[...], preferred_element_type=jnp.float32) + br_ref[...]


def _project(x, W_l, b_l, W_r, b_r):
    blk = 1000
    return pl.pallas_call(
        _proj_body,
        grid=(N // blk,),
        in_specs=[
            pl.BlockSpec((blk, D), lambda i: (i, 0)),
            pl.BlockSpec((D, HC), lambda i: (0, 0)),
            pl.BlockSpec((D, HC), lambda i: (0, 0)),
            pl.BlockSpec((1, HC), lambda i: (0, 0)),
            pl.BlockSpec((1, HC), lambda i: (0, 0)),
        ],
        out_specs=[
            pl.BlockSpec((blk, HC), lambda i: (i, 0)),
            pl.BlockSpec((blk, HC), lambda i: (i, 0)),
        ],
        out_shape=[
            jax.ShapeDtypeStruct((N, HC), jnp.float32),
            jax.ShapeDtypeStruct((N, HC), jnp.float32),
        ],
    )(x, W_l, W_r, b_l.reshape(1, HC), b_r.reshape(1, HC))


def _sca_body(pk_h, xl_h, xr_h, we_h, att_h,
              acc_o, lw_o, ea_o,
              acc_sp,
              pk0, pk1,
              xl0, xl1, xr0, xr1, m_v, ia_v, ea0, ea1,
              lw_t, zb, wa_v,
              isem, gsem0, gsem1, ssem, easem):
    cid = lax.axis_index("c")
    sid = lax.axis_index("s")
    wid = cid * NS + sid

    pks = (pk0, pk1)
    xls, xrs = (xl0, xl1), (xr0, xr1)
    eas = (ea0, ea1)
    gsems = (gsem0, gsem1)

    zero16 = jnp.zeros((16,), jnp.float32)

    def z1(t, _):
        zb[t // 8, pl.ds((t % 8) * 16, 16)] = zero16
        return 0
    lax.fori_loop(0, 32 * 8, z1, 0)

    def z2(k, _):
        lw_t[pl.ds(k * 16, 16)] = zero16
        return 0
    lax.fori_loop(0, NP // 16, z2, 0)

    row0 = pl.multiple_of(sid * (NA // NS), 8)
    for k in range(NA // NS // 32):
        pltpu.sync_copy(zb, acc_sp.at[pl.ds(row0 + k * 32, 32)])

    pltpu.sync_copy(we_h, wa_v.at[0])
    pltpu.sync_copy(att_h, wa_v.at[1])

    plsc.subcore_barrier()

    wev = tuple(wa_v[0, pl.ds(v * 16, 16)] for v in range(8))
    atv = tuple(wa_v[1, pl.ds(v * 16, 16)] for v in range(8))
    lane = lax.iota(jnp.int32, 16)
    lane0 = lane == 0
    masks = tuple((lane == h).astype(jnp.float32) for h in range(H))

    def idx_start(j, q):
        base = pl.multiple_of((wid * NBA + j) * 3 * BA, 8)
        pltpu.async_copy(pk_h.at[pl.ds(base, 3 * BA)], pks[q], isem)

    def idx_wait(j, q):
        base = pl.multiple_of((wid * NBA + j) * 3 * BA, 8)
        pltpu.make_async_copy(pk_h.at[pl.ds(base, 3 * BA)], pks[q], isem).wait()

    def gather_start(q):
        pltpu.async_copy(xl_h.at[pks[q].at[pl.ds(0, BA)]], xls[q], gsems[q])
        pltpu.async_copy(xr_h.at[pks[q].at[pl.ds(BA, BA)]], xrs[q], gsems[q])

    def gather_wait(q):
        pltpu.make_async_copy(xl_h.at[pks[q].at[pl.ds(0, BA)]], xls[q],
                              gsems[q]).wait()
        pltpu.make_async_copy(xr_h.at[pks[q].at[pl.ds(BA, BA)]], xrs[q],
                              gsems[q]).wait()

    def scatter_start():
        pltpu.async_copy(m_v, acc_sp.at[ia_v], ssem, add=True)

    def scatter_wait():
        pltpu.make_async_copy(m_v, acc_sp.at[ia_v], ssem).wait()

    def compute(j, q, carry):
        def group_body(g, carry):
            wev, atv, lane0 = carry
            d16 = pks[q][pl.ds(BA + g * 16, 16)]
            w16 = plsc.bitcast(pks[q][pl.ds(2 * BA + g * 16, 16)], jnp.float32)
            ia_v[pl.ds(g * 16, 16)] = jnp.where(d16 < NA, d16, 0)
            for k in range(16):
                dkv = jnp.full((16,), d16[k], jnp.int32)
                wkv = jnp.full((16,), w16[k], jnp.float32)
                cur = plsc.load_gather(lw_t, [dkv])
                plsc.store_scatter(lw_t, [dkv], jnp.maximum(cur, wkv),
                                   mask=lane0)
            for k in range(16):
                i = g * 16 + k
                w_s = w16[k]
                ind = (d16[k] < NA).astype(jnp.float32)
                ps = []
                for v in range(8):
                    xlv = xls[q][i, pl.ds(v * 16, 16)]
                    xrv = xrs[q][i, pl.ds(v * 16, 16)]
                    u = xlv + xrv + w_s * wev[v]
                    t = jnp.maximum(u, _SLOPE * u) * atv[v]
                    ps.append((t, xlv))
                evecs = []
                pack = jnp.zeros((16,), jnp.float32)
                for h in range(H):
                    a_h = jnp.sum(ps[2 * h][0] + ps[2 * h + 1][0])
                    e_h = jnp.exp(jnp.full((16,), a_h))
                    evecs.append(e_h)
                    pack = pack + e_h * masks[h]
                eas[q][pl.ds(i * 16, 16)] = pack
                for v in range(8):
                    m_v[i, pl.ds(v * 16, 16)] = ps[v][1] * (evecs[v // 2] * ind)
            return (wev, atv, lane0)

        return lax.fori_loop(0, BA // 16, group_body, carry)

    def step(j, q, carry, first, last, ea_first=False):
        gather_wait(q)
        if not last:
            idx_wait(j + 1, 1 - q)
            gather_start(1 - q)
        if not first:
            scatter_wait()
        if not (first or ea_first):
            earp = pl.multiple_of((wid * NBA + (j - 2)) * BA * 16, 8)
            pltpu.make_async_copy(eas[q], ea_o.at[pl.ds(earp, BA * 16)],
                                  easem).wait()
        carry = compute(j, q, carry)
        ear = pl.multiple_of((wid * NBA + j) * BA * 16, 8)
        pltpu.async_copy(eas[q], ea_o.at[pl.ds(ear, BA * 16)], easem)
        if not last:
            @pl.when(j + 2 < NBA)
            def _():
                idx_start(j + 2, q)
        scatter_start()
        return carry

    carry = (wev, atv, lane0)
    idx_start(0, 0)
    idx_wait(0, 0)
    gather_start(0)
    idx_start(1, 1)
    carry = step(0, 0, carry, True, False)
    carry = step(1, 1, carry, False, False, ea_first=True)

    def super_body(t, carry):
        j0 = t * 2
        carry = step(j0, 0, carry, False, False)
        carry = step(j0 + 1, 1, carry, False, False)
        return carry

    carry = lax.fori_loop(1, (NBA - 1) // 2, super_body, carry)
    carry = step(NBA - 1, 0, carry, False, True)
    scatter_wait()
    for jt in (NBA - 2, NBA - 1):
        earp = pl.multiple_of((wid * NBA + jt) * BA * 16, 8)
        pltpu.make_async_copy(eas[jt % 2], ea_o.at[pl.ds(earp, BA * 16)],
                              easem).wait()

    plsc.subcore_barrier()

    orow = pl.multiple_of(cid * NA + sid * (NA // NS), 8)
    pltpu.sync_copy(acc_sp.at[pl.ds(row0, NA // NS)],
                    acc_o.at[pl.ds(orow, NA // NS)])
    pltpu.sync_copy(lw_t, lw_o.at[pl.ds(pl.multiple_of(wid * NP, 8), NP)])


def _sca(pk, xl, xr, we_row, att_row):
    mesh = plsc.VectorSubcoreMesh(core_axis_name="c", subcore_axis_name="s")
    f = functools.partial(
        pl.kernel,
        out_type=[
            jax.ShapeDtypeStruct((NC * NA, HC), jnp.float32),
            jax.ShapeDtypeStruct((NW * NP,), jnp.float32),
            jax.ShapeDtypeStruct((E * 16,), jnp.float32),
        ],
        mesh=mesh,
        compiler_params=pltpu.CompilerParams(needs_layout_passes=False),
        scratch_types=[
            pltpu.VMEM_SHARED((NA, HC), jnp.float32),
            pltpu.VMEM((3 * BA,), jnp.int32),
            pltpu.VMEM((3 * BA,), jnp.int32),
            pltpu.VMEM((BA, HC), jnp.float32),
            pltpu.VMEM((BA, HC), jnp.float32),
            pltpu.VMEM((BA, HC), jnp.float32),
            pltpu.VMEM((BA, HC), jnp.float32),
            pltpu.VMEM((BA, HC), jnp.float32),
            pltpu.VMEM((BA,), jnp.int32),
            pltpu.VMEM((BA * 16,), jnp.float32),
            pltpu.VMEM((BA * 16,), jnp.float32),
            pltpu.VMEM((NP,), jnp.float32),
            pltpu.VMEM((32, HC), jnp.float32),
            pltpu.VMEM((2, HC), jnp.float32),
            pltpu.SemaphoreType.DMA,
            pltpu.SemaphoreType.DMA,
            pltpu.SemaphoreType.DMA,
            pltpu.SemaphoreType.DMA,
            pltpu.SemaphoreType.DMA,
        ],
    )(_sca_body)
    return f(pk, xl, xr, we_row, att_row)


def _scb_body(pk_h, xl_h, ea_h,
              acc_o, den_o,
              acc_sp, den_sp,
              pk0, pk1, ib_v, dn_v,
              xl0, xl1, m_v, dp_v, ea0, ea1, zb,
              isem, gsem0, gsem1, ssem):
    cid = lax.axis_index("c")
    sid = lax.axis_index("s")
    wid = cid * NS + sid

    pks = (pk0, pk1)
    xls, eas = (xl0, xl1), (ea0, ea1)
    gsems = (gsem0, gsem1)

    zero16 = jnp.zeros((16,), jnp.float32)

    def z1(t, _):
        zb[t // 8, pl.ds((t % 8) * 16, 16)] = zero16
        return 0
    lax.fori_loop(0, 32 * 8, z1, 0)

    rowb = pl.multiple_of(sid * (NBR // NS), 8)
    for k in range(NBR // NS // 32):
        pltpu.sync_copy(zb, acc_sp.at[pl.ds(rowb + k * 32, 32)])
    rowd = pl.multiple_of(sid * (ND8 // NS), 8)
    for k in range(2):
        pltpu.sync_copy(zb, den_sp.at[pl.ds(rowd + k * 32, 32)])
    pltpu.sync_copy(zb.at[pl.ds(0, 16)], den_sp.at[pl.ds(rowd + 64, 16)])

    plsc.subcore_barrier()

    lane = lax.iota(jnp.int32, 16)

    def idx_start(j, q):
        base = pl.multiple_of((wid * NB + j) * 3 * B, 8)
        pltpu.async_copy(pk_h.at[pl.ds(base, 2 * B)], pks[q], isem)

    def idx_wait(j, q):
        base = pl.multiple_of((wid * NB + j) * 3 * B, 8)
        pltpu.make_async_copy(pk_h.at[pl.ds(base, 2 * B)], pks[q], isem).wait()

    def gather_start(j, q):
        pltpu.async_copy(xl_h.at[pks[q].at[pl.ds(0, B)]], xls[q], gsems[q])
        ear = pl.multiple_of((wid * NB + j) * B * 16, 8)
        pltpu.async_copy(ea_h.at[pl.ds(ear, B * 16)], eas[q], gsems[q])

    def gather_wait(j, q):
        pltpu.make_async_copy(xl_h.at[pks[q].at[pl.ds(0, B)]], xls[q],
                              gsems[q]).wait()
        ear = pl.multiple_of((wid * NB + j) * B * 16, 8)
        pltpu.make_async_copy(ea_h.at[pl.ds(ear, B * 16)], eas[q], gsems[q]).wait()

    def scatter_start():
        pltpu.async_copy(m_v, acc_sp.at[ib_v], ssem, add=True)
        pltpu.async_copy(dp_v, den_sp.at[dn_v], ssem, add=True)

    def scatter_wait():
        pltpu.make_async_copy(m_v, acc_sp.at[ib_v], ssem).wait()
        pltpu.make_async_copy(dp_v, den_sp.at[dn_v], ssem).wait()

    def compute(q, carry):
        def group_body(g, carry):
            lane, = carry
            d16 = pks[q][pl.ds(B + g * 16, 16)]
            ib_v[pl.ds(g * 16, 16)] = jnp.where(d16 >= NA, d16 - NA, 0)
            dn_v[pl.ds(g * 16, 16)] = d16 // 8
            for k in range(16):
                i = g * 16 + k
                ind = (d16[k] >= NA).astype(jnp.float32)
                m8 = (d16[k] % 8) * 16
                pack = jnp.zeros((16,), jnp.float32)
                for h in range(H):
                    e_h = eas[q][pl.ds(i * 64 + h * 16, 16)]
                    pack = pack + e_h * (lane == h).astype(jnp.float32)
                    sl = e_h * ind
                    m_v[i, pl.ds(h * 32, 16)] = (
                        xls[q][i, pl.ds(h * 32, 16)] * sl)
                    m_v[i, pl.ds(h * 32 + 16, 16)] = (
                        xls[q][i, pl.ds(h * 32 + 16, 16)] * sl)
                for r in range(8):
                    dp_v[i, pl.ds(r * 16, 16)] = zero16
                dp_v[i, pl.ds(m8, 16)] = pack
            return (lane,)

        return lax.fori_loop(0, B // 16, group_body, carry)

    def step(j, q, carry, first, last):
        gather_wait(j, q)
        if not last:
            idx_wait(j + 1, 1 - q)
            gather_start(j + 1, 1 - q)
        if not first:
            scatter_wait()
        carry = compute(q, carry)
        if not last:
            @pl.when(j + 2 < NB)
            def _():
                idx_start(j + 2, q)
        scatter_start()
        return carry

    carry = (lane,)
    idx_start(0, 0)
    idx_wait(0, 0)
    gather_start(0, 0)
    idx_start(1, 1)
    carry = step(0, 0, carry, True, False)
    carry = step(1, 1, carry, False, False)

    def super_body(t, carry):
        j0 = t * 2
        carry = step(j0, 0, carry, False, False)
        carry = step(j0 + 1, 1, carry, False, False)
        return carry

    carry = lax.fori_loop(1, (NB - 1) // 2, super_body, carry)
    carry = step(NB - 1, 0, carry, False, True)
    scatter_wait()

    plsc.subcore_barrier()

    orow = pl.multiple_of(cid * NBR + sid * (NBR // NS), 8)
    pltpu.sync_copy(acc_sp.at[pl.ds(rowb, NBR // NS)],
                    acc_o.at[pl.ds(orow, NBR // NS)])
    drow = pl.multiple_of(cid * ND8 + sid * (ND8 // NS), 8)
    pltpu.sync_copy(den_sp.at[pl.ds(rowd, ND8 // NS)],
                    den_o.at[pl.ds(drow, ND8 // NS)])


def _scb(pk, xl, ea):
    mesh = plsc.VectorSubcoreMesh(core_axis_name="c", subcore_axis_name="s")
    f = functools.partial(
        pl.kernel,
        out_type=[
            jax.ShapeDtypeStruct((NC * NBR, HC), jnp.float32),
            jax.ShapeDtypeStruct((NC * ND8, HC), jnp.float32),
        ],
        mesh=mesh,
        compiler_params=pltpu.CompilerParams(needs_layout_passes=False),
        scratch_types=[
            pltpu.VMEM_SHARED((NBR, HC), jnp.float32),
            pltpu.VMEM_SHARED((ND8, HC), jnp.float32),
            pltpu.VMEM((2 * B,), jnp.int32),
            pltpu.VMEM((2 * B,), jnp.int32),
            pltpu.VMEM((B,), jnp.int32),
            pltpu.VMEM((B,), jnp.int32),
            pltpu.VMEM((B, HC), jnp.float32),
            pltpu.VMEM((B, HC), jnp.float32),
            pltpu.VMEM((B, HC), jnp.float32),
            pltpu.VMEM((B, HC), jnp.float32),
            pltpu.VMEM((B * 16,), jnp.float32),
            pltpu.VMEM((B * 16,), jnp.float32),
            pltpu.VMEM((32, HC), jnp.float32),
            pltpu.SemaphoreType.DMA,
            pltpu.SemaphoreType.DMA,
            pltpu.SemaphoreType.DMA,
            pltpu.SemaphoreType.DMA,
        ],
    )(_scb_body)
    return f(pk, xl, ea)


def _final_body(a0_ref, a1_ref, d0_ref, d1_ref, lw_ref,
                xl_ref, xr_ref, we_ref, attb_ref, b4_ref, bias_ref, o_ref):
    xl = xl_ref[...]
    lw = jnp.max(lw_ref[...], axis=1)[:, None]
    u = xl + xr_ref[...] + lw * we_ref[...]
    t = jnp.maximum(u, _SLOPE * u)
    eb = jnp.exp(jnp.dot(t, attb_ref[...], preferred_element_type=jnp.float32))
    den_b = jnp.dot(d0_ref[...] + d1_ref[...], b4_ref[...],
                    preferred_element_type=jnp.float32) + eb
    num = a0_ref[...] + a1_ref[...] + xl * eb
    o_ref[...] = num / (den_b + 1e-16) + bias_ref[...]


def _finalize(nrows, acc2, den8, lw_nt, xl, xr, we_row, att_bcast, b4, bias):
    blk = 512
    g = nrows // blk
    return pl.pallas_call(
        _final_body,
        grid=(g,),
        in_specs=[
            pl.BlockSpec((blk, HC), lambda i: (i, 0)),
            pl.BlockSpec((blk, HC), lambda i, g=g: (i + g, 0)),
            pl.BlockSpec((blk, 8), lambda i: (i, 0)),
            pl.BlockSpec((blk, 8), lambda i, g=g: (i + g, 0)),
            pl.BlockSpec((blk, NW), lambda i: (i, 0)),
            pl.BlockSpec((blk, HC), lambda i: (i, 0)),
            pl.BlockSpec((blk, HC), lambda i: (i, 0)),
            pl.BlockSpec((1, HC), lambda i: (0, 0)),
            pl.BlockSpec((HC, HC), lambda i: (0, 0)),
            pl.BlockSpec((8, HC), lambda i: (0, 0)),
            pl.BlockSpec((1, HC), lambda i: (0, 0)),
        ],
        out_specs=pl.BlockSpec((blk, HC), lambda i: (i, 0)),
        out_shape=jax.ShapeDtypeStruct((nrows, HC), jnp.float32),
    )(acc2, acc2, den8, den8, lw_nt, xl, xr,
      we_row.reshape(1, HC), att_bcast, b4, bias.reshape(1, HC))


def kernel(x, edge_index, edge_weight, W_l, b_l, W_r, b_r, W_e, att, bias):
    wbits = jax.lax.bitcast_convert_type(edge_weight, jnp.int32)
    pk = (jnp.stack([edge_index[0].reshape(E // B, B),
                     edge_index[1].reshape(E // B, B),
                     wbits.reshape(E // B, B)], axis=1)
          .reshape(3 * E))

    xl, xr = _project(x, W_l, b_l, W_r, b_r)

    we_row = W_e.reshape(HC)
    att_flat = att.reshape(HC)
    accA, lw_flat, ea = _sca(pk, xl, xr, we_row, att_flat)
    accB, den8 = _scb(pk, xl, ea)

    ci = jnp.arange(HC, dtype=jnp.int32) // C
    att_bcast = jnp.where(ci[:, None] == ci[None, :], att_flat[:, None], 0.0)
    hh = jnp.arange(8, dtype=jnp.int32)
    b4 = (hh[:, None] == ci[None, :]).astype(jnp.float32)

    den4 = den8.reshape(NC * ND8, 8, 16)[:, :, :4]
    den4 = den4.reshape(NC, NP, 4)
    den4 = jnp.pad(den4, ((0, 0), (0, 0), (0, 4)))

    lw_nt = lw_flat.reshape(NW, NP)[:, :N].T
    lw_pad = jnp.pad(lw_nt, ((0, NP - N), (0, 0)))
    xl_pad = jnp.pad(xl, ((0, NP - N), (0, 0)))
    xr_pad = jnp.pad(xr, ((0, NP - N), (0, 0)))

    denA = jnp.concatenate([den4[0, :NA], den4[1, :NA]], 0)
    outA = _finalize(NA, accA, denA, lw_pad[:NA], xl_pad[:NA], xr_pad[:NA],
                     we_row, att_bcast, b4, bias)

    denB = jnp.concatenate([den4[0, NA:], den4[1, NA:]], 0)
    outB = _finalize(NBR, accB, denB, lw_pad[NA:], xl_pad[NA:], xr_pad[NA:],
                     we_row, att_bcast, b4, bias)

    return jnp.concatenate([outA, outB[:N - NA]], axis=0)
